# Initial kernel scaffold; baseline (speedup 1.0000x reference)
import jax
import jax.numpy as jnp
from jax.experimental import pallas as pl


def kernel(idx, alpha):
    raise NotImplementedError("write your pallas kernel here")


# speedup vs baseline: 137.7288x; 137.7288x over previous
import functools

import jax
import jax.numpy as jnp
from jax import lax
from jax.experimental import pallas as pl
from jax.experimental.pallas import tpu as pltpu
from jax.experimental.pallas import tpu_sc as plsc

_NC = 2
_NS = 16
_NW = _NC * _NS


def _softmax_body(alpha_ref, out_ref):
    a = alpha_ref[...]
    m = jnp.max(a)
    e = jnp.exp(a - m)
    out_ref[...] = e / jnp.sum(e)


def _softmax_table(alpha_padded_2d):
    return pl.pallas_call(
        _softmax_body,
        out_shape=jax.ShapeDtypeStruct(alpha_padded_2d.shape, jnp.float32),
    )(alpha_padded_2d)


def _sc_gather_body(nchunks, chunk, b_per_w, table_hbm, idx_hbm, out_hbm,
                    idx_v, rows_v, sem):
    wid = lax.axis_index("s") * _NC + lax.axis_index("c")
    base = wid * b_per_w
    for c in range(nchunks):
        off = base + c * chunk
        pltpu.sync_copy(idx_hbm.at[pl.ds(off, chunk)], idx_v)
        pltpu.async_copy(table_hbm.at[idx_v], rows_v, sem).wait()
        pltpu.sync_copy(rows_v, out_hbm.at[pl.ds(off, chunk)])


@functools.partial(jax.jit, static_argnames=())
def kernel(idx, alpha):
    batch, hist = idx.shape
    n = alpha.shape[0]

    n_pad = (-n) % 128
    ap = jnp.pad(alpha, (0, n_pad), constant_values=-jnp.inf)
    table = _softmax_table(ap.reshape(-1, 128)).reshape(-1)

    bflat = batch * hist
    assert bflat % (8 * _NW) == 0
    b_per_w = bflat // _NW
    chunk = b_per_w
    nchunks = 1
    while chunk * 8 > 384 * 1024 or chunk % 8 != 0:
        nchunks += 1
        while b_per_w % nchunks != 0:
            nchunks += 1
        chunk = b_per_w // nchunks

    mesh = plsc.VectorSubcoreMesh(core_axis_name="c", subcore_axis_name="s")
    gather = pl.kernel(
        functools.partial(_sc_gather_body, nchunks, chunk, b_per_w),
        out_type=jax.ShapeDtypeStruct((bflat,), jnp.float32),
        mesh=mesh,
        scratch_types=[
            pltpu.VMEM((chunk,), jnp.int32),
            pltpu.VMEM((chunk,), jnp.float32),
            pltpu.SemaphoreType.DMA,
        ],
    )
    out_flat = gather(table, idx.reshape(-1))
    return out_flat.reshape(batch, hist)
